# Initial kernel scaffold; baseline (speedup 1.0000x reference)
import jax
import jax.numpy as jnp
from jax.experimental import pallas as pl


def kernel(pro1_x, pro1_edge_index, pro1_batch, pro2_x, pro2_edge_index, pro2_batch, mas1_straight, mas1_flipped, mas2_straight, mas2_flipped, W1, b1, fc1_W, fc1_b, W2, b2, fc2_W, fc2_b, final_W, final_b):
    raise NotImplementedError("write your pallas kernel here")


# speedup vs baseline: 18.2301x; 18.2301x over previous
import functools

import jax
import jax.numpy as jnp
from jax import lax
from jax.experimental import pallas as pl
from jax.experimental.pallas import tpu as pltpu
from jax.experimental.pallas import tpu_sc as plsc

_NC = 2
_NS = 16
_CHUNK = 128
_G = 64
_BN = 1000


@functools.lru_cache(maxsize=None)
def _make_deg_kernel(Ep, R):
    cpt = Ep // _NS // _CHUNK
    rpt = R // _NS
    mesh = plsc.VectorSubcoreMesh(core_axis_name="c", subcore_axis_name="s")

    @functools.partial(
        pl.kernel,
        out_type=[jax.ShapeDtypeStruct((R,), jnp.float32)] * 2,
        mesh=mesh,
        scratch_types=[
            pltpu.VMEM((_CHUNK,), jnp.int32),
            pltpu.VMEM((_CHUNK,), jnp.float32),
            pltpu.VMEM((rpt,), jnp.float32),
            pltpu.VMEM_SHARED((R,), jnp.float32),
        ],
    )
    def k(dst1_hbm, dst2_hbm, deg1_hbm, deg2_hbm,
          dst_v, ones_v, zbuf_v, deg_sh):
        c = lax.axis_index("c")
        s = lax.axis_index("s")
        for g in range(_CHUNK // 16):
            ones_v[pl.ds(g * 16, 16)] = jnp.ones((16,), jnp.float32)

        def zfill(i, carry):
            zbuf_v[pl.ds(i * 16, 16)] = jnp.zeros((16,), jnp.float32)
            return carry
        lax.fori_loop(0, rpt // 16, zfill, 0)
        pltpu.sync_copy(zbuf_v, deg_sh.at[pl.ds(s * rpt, rpt)])
        plsc.subcore_barrier()

        def scatter(dst_hbm):
            def body(ki, carry):
                base = s * (cpt * _CHUNK) + ki * _CHUNK
                pltpu.sync_copy(dst_hbm.at[pl.ds(base, _CHUNK)], dst_v)
                pltpu.sync_copy(ones_v, deg_sh.at[dst_v], add=True)
                return carry
            lax.fori_loop(0, cpt, body, 0)

        @pl.when(c == 0)
        def _():
            scatter(dst1_hbm)

        @pl.when(c == 1)
        def _():
            scatter(dst2_hbm)

        plsc.subcore_barrier()

        @pl.when(c == 0)
        def _():
            pltpu.sync_copy(deg_sh.at[pl.ds(s * rpt, rpt)],
                            deg1_hbm.at[pl.ds(s * rpt, rpt)])

        @pl.when(c == 1)
        def _():
            pltpu.sync_copy(deg_sh.at[pl.ds(s * rpt, rpt)],
                            deg2_hbm.at[pl.ds(s * rpt, rpt)])

    return k


def _y_body(x1_ref, w1_ref, deg1_ref, x2_ref, w2_ref, deg2_ref,
            y1_ref, y2_ref):
    d1 = lax.rsqrt(deg1_ref[...] + 1.0)
    y1_ref[...] = d1 * jnp.dot(x1_ref[...], w1_ref[...],
                               preferred_element_type=jnp.float32)
    d2 = lax.rsqrt(deg2_ref[...] + 1.0)
    y2_ref[...] = d2 * jnp.dot(x2_ref[...], w2_ref[...],
                               preferred_element_type=jnp.float32)


def _y_call(x1, w1, deg1, x2, w2, deg2):
    n, d = x1.shape
    nb = n // _BN
    row_blk = pl.BlockSpec((_BN, d), lambda i: (i, 0))
    deg_blk = pl.BlockSpec((_BN, 1), lambda i: (i, 0))
    full = pl.BlockSpec((d, d), lambda i: (0, 0))
    return pl.pallas_call(
        _y_body,
        grid=(nb,),
        in_specs=[row_blk, full, deg_blk, row_blk, full, deg_blk],
        out_specs=[row_blk, row_blk],
        out_shape=[jax.ShapeDtypeStruct((n, d), jnp.float32)] * 2,
    )(x1, w1, deg1, x2, w2, deg2)


@functools.lru_cache(maxsize=None)
def _make_edge_kernel(N, Ep, R):
    ept = Ep // _NS
    cpt = ept // _CHUNK
    rpt = R // _NS
    mesh = plsc.VectorSubcoreMesh(core_axis_name="c", subcore_axis_name="s")

    @functools.partial(
        pl.kernel,
        out_type=[jax.ShapeDtypeStruct((R, 128), jnp.float32)] * 2,
        mesh=mesh,
        scratch_types=[
            pltpu.VMEM((_CHUNK,), jnp.int32),
            pltpu.VMEM((_CHUNK,), jnp.int32),
            pltpu.VMEM((_CHUNK, 128), jnp.float32),
            pltpu.VMEM_SHARED((R, 128), jnp.float32),
            pltpu.SemaphoreType.DMA,
        ],
    )
    def k(y1_hbm, y2_hbm, src1_hbm, dst1_hbm, src2_hbm, dst2_hbm, zeros_hbm,
          agg1_hbm, agg2_hbm, src_v, dst_v, rows_v, agg_sh, sem):
        c = lax.axis_index("c")
        s = lax.axis_index("s")
        pltpu.sync_copy(zeros_hbm, agg_sh.at[pl.ds(s * rpt, rpt)])
        plsc.subcore_barrier()

        def run(y_hbm, src_hbm, dst_hbm):
            def body(ki, carry):
                base = s * ept + ki * _CHUNK
                pltpu.sync_copy(src_hbm.at[pl.ds(base, _CHUNK)], src_v)
                pltpu.sync_copy(dst_hbm.at[pl.ds(base, _CHUNK)], dst_v)
                pltpu.async_copy(y_hbm.at[src_v], rows_v, sem).wait()
                pltpu.sync_copy(rows_v, agg_sh.at[dst_v], add=True)
                return carry
            lax.fori_loop(0, cpt, body, 0)

        @pl.when(c == 0)
        def _():
            run(y1_hbm, src1_hbm, dst1_hbm)

        @pl.when(c == 1)
        def _():
            run(y2_hbm, src2_hbm, dst2_hbm)

        plsc.subcore_barrier()

        @pl.when(c == 0)
        def _():
            pltpu.sync_copy(agg_sh.at[pl.ds(s * rpt, rpt)],
                            agg1_hbm.at[pl.ds(s * rpt, rpt)])

        @pl.when(c == 1)
        def _():
            pltpu.sync_copy(agg_sh.at[pl.ds(s * rpt, rpt)],
                            agg2_hbm.at[pl.ds(s * rpt, rpt)])

    return k


def _leaky(v):
    return jnp.where(v >= 0, v, 0.01 * v)


def _final_body(batch1_ref, agg1_ref, y1_ref, deg1_ref, b1_ref,
                fc1w_ref, fc1b_ref,
                batch2_ref, agg2_ref, y2_ref, deg2_ref, b2_ref,
                fc2w_ref, fc2b_ref,
                fw_ref, fb_ref, out_ref, p1, c1, p2, c2):
    i = pl.program_id(0)

    @pl.when(i == 0)
    def _():
        p1[...] = jnp.zeros_like(p1)
        c1[...] = jnp.zeros_like(c1)
        p2[...] = jnp.zeros_like(p2)
        c2[...] = jnp.zeros_like(c2)

    def acc(batch_ref, agg_ref, y_ref, deg_ref, b_ref, p, c):
        dinv = lax.rsqrt(deg_ref[...] + 1.0)
        vals = _leaky(dinv * (agg_ref[...] + y_ref[...]) + b_ref[...])
        bt = batch_ref[0]
        oh = (bt == lax.broadcasted_iota(jnp.int32, (_G, _BN), 0)
              ).astype(jnp.float32)
        p[...] += jnp.dot(oh, vals, preferred_element_type=jnp.float32)
        c[...] += jnp.sum(oh, axis=1, keepdims=True)

    acc(batch1_ref, agg1_ref, y1_ref, deg1_ref, b1_ref, p1, c1)
    acc(batch2_ref, agg2_ref, y2_ref, deg2_ref, b2_ref, p2, c2)

    @pl.when(i == pl.num_programs(0) - 1)
    def _():
        m1 = p1[...] / jnp.clip(c1[...], 1.0, None)
        h1 = _leaky(jnp.dot(m1, fc1w_ref[...],
                            preferred_element_type=jnp.float32) + fc1b_ref[...])
        m2 = p2[...] / jnp.clip(c2[...], 1.0, None)
        h2 = _leaky(jnp.dot(m2, fc2w_ref[...],
                            preferred_element_type=jnp.float32) + fc2b_ref[...])
        comb = jnp.concatenate([h1, h2], axis=1)
        out_ref[...] = jnp.dot(comb, fw_ref[...],
                               preferred_element_type=jnp.float32) + fb_ref[...]


def _final_call(batch1, agg1, y1, deg1, b1, fc1w, fc1b,
                batch2, agg2, y2, deg2, b2, fc2w, fc2b, fw, fb):
    n, d = y1.shape
    nb = n // _BN
    out2 = fw.shape[0]
    row_blk = pl.BlockSpec((_BN, d), lambda i: (i, 0))
    deg_blk = pl.BlockSpec((_BN, 1), lambda i: (i, 0))
    batch_blk = pl.BlockSpec((1, 1, _BN), lambda i: (i, 0, 0))
    bias_blk = pl.BlockSpec((1, d), lambda i: (0, 0))
    w_blk = pl.BlockSpec((d, d), lambda i: (0, 0))
    fw_blk = pl.BlockSpec((out2, 1), lambda i: (0, 0))
    fb_blk = pl.BlockSpec((1, 1), lambda i: (0, 0))
    out_blk = pl.BlockSpec((_G, 1), lambda i: (0, 0))
    return pl.pallas_call(
        _final_body,
        grid=(nb,),
        in_specs=[batch_blk, row_blk, row_blk, deg_blk, bias_blk, w_blk,
                  bias_blk,
                  batch_blk, row_blk, row_blk, deg_blk, bias_blk, w_blk,
                  bias_blk, fw_blk, fb_blk],
        out_specs=out_blk,
        out_shape=jax.ShapeDtypeStruct((_G, 1), jnp.float32),
        scratch_shapes=[pltpu.VMEM((_G, d), jnp.float32),
                        pltpu.VMEM((_G, 1), jnp.float32),
                        pltpu.VMEM((_G, d), jnp.float32),
                        pltpu.VMEM((_G, 1), jnp.float32)],
    )(batch1, agg1, y1, deg1, b1, fc1w, fc1b,
      batch2, agg2, y2, deg2, b2, fc2w, fc2b, fw, fb)


def kernel(pro1_x, pro1_edge_index, pro1_batch, pro2_x, pro2_edge_index,
           pro2_batch, mas1_straight, mas1_flipped, mas2_straight,
           mas2_flipped, W1, b1, fc1_W, fc1_b, W2, b2, fc2_W, fc2_b,
           final_W, final_b):
    n, d = pro1_x.shape
    e = pro1_edge_index.shape[1]

    ep = -(-e // (_NS * _CHUNK)) * (_NS * _CHUNK)
    pad = ep - e
    r = -(-(n + 1) // (_NS * 16)) * (_NS * 16)

    def pad_edges(edge_index):
        src = jnp.concatenate(
            [edge_index[0], jnp.zeros((pad,), jnp.int32)])
        dst = jnp.concatenate(
            [edge_index[1], jnp.full((pad,), n, jnp.int32)])
        return src, dst

    src1, dst1 = pad_edges(pro1_edge_index)
    src2, dst2 = pad_edges(pro2_edge_index)

    zeros128 = jnp.zeros((r // _NS, d), jnp.float32)

    degf_1, degf_2 = _make_deg_kernel(ep, r)(dst1, dst2)
    deg1 = degf_1[:n, None]
    deg2 = degf_2[:n, None]

    y1, y2 = _y_call(pro1_x, W1, deg1, pro2_x, W2, deg2)

    agg1p, agg2p = _make_edge_kernel(n, ep, r)(
        y1, y2, src1, dst1, src2, dst2, zeros128)
    agg1 = agg1p[:n]
    agg2 = agg2p[:n]

    batch1 = pro1_batch.reshape(n // _BN, 1, _BN)
    batch2 = pro2_batch.reshape(n // _BN, 1, _BN)

    return _final_call(
        batch1, agg1, y1, deg1, b1.reshape(1, d), fc1_W, fc1_b.reshape(1, d),
        batch2, agg2, y2, deg2, b2.reshape(1, d), fc2_W, fc2_b.reshape(1, d),
        final_W, final_b.reshape(1, 1))
